# baseline (speedup 1.0000x reference)
import functools
import jax
import jax.numpy as jnp
from jax import lax
from jax.experimental import pallas as pl
from jax.experimental.pallas import tpu as pltpu
from jax.experimental.pallas import tpu_sc as plsc

_ROWS = 8 * 576
_D = 8192
_NW = 32
_RPW = _ROWS // _NW
_CHUNK = 8
_NCHUNK = _RPW // _CHUNK


def _make_sc_copy():
    mesh = plsc.VectorSubcoreMesh(core_axis_name="c", subcore_axis_name="s")

    @functools.partial(
        pl.kernel,
        mesh=mesh,
        out_type=jax.ShapeDtypeStruct((_ROWS, _D), jnp.float32),
        scratch_types=[pltpu.VMEM((_CHUNK, _D), jnp.float32)],
    )
    def k(x_hbm, out_hbm, buf_v):
        wid = lax.axis_index("s") * 2 + lax.axis_index("c")
        base = wid * _RPW

        def body(i, carry):
            row0 = base + i * _CHUNK
            pltpu.sync_copy(x_hbm.at[pl.ds(row0, _CHUNK)], buf_v)
            pltpu.sync_copy(buf_v, out_hbm.at[pl.ds(row0, _CHUNK)])
            return carry

        lax.fori_loop(0, _NCHUNK, body, 0)

    return k


_sc_copy = _make_sc_copy()


def kernel(x, temp_log):
    xf = x.reshape(_ROWS, _D)
    out = _sc_copy(xf)
    return out.reshape(x.shape), jnp.exp(temp_log)


# speedup vs baseline: 1.4103x; 1.4103x over previous
import jax
import jax.numpy as jnp
from jax import lax
from jax.experimental import pallas as pl
from jax.experimental.pallas import tpu as pltpu

_B0, _B1, _D = 8, 576, 8192
_ROWS = _B0 * _B1
_R = 128
_NBUF = 4
_NCHUNK = _ROWS // _R
_NROUND = _NCHUNK // _NBUF


def _body(tl_ref, x_hbm, o_hbm, in_buf, out_buf, in_sem, out_sem):
    inv_temp = jnp.exp(-tl_ref[0, 0])

    def in_copy(g, s):
        return pltpu.make_async_copy(
            x_hbm.at[pl.ds(g * _R, _R)], in_buf.at[s], in_sem.at[s])

    def out_copy(g, s):
        return pltpu.make_async_copy(
            out_buf.at[s], o_hbm.at[pl.ds(g * _R, _R)], out_sem.at[s])

    for s in range(_NBUF):
        in_copy(s, s).start()

    def round_body(r, carry):
        for s in range(_NBUF):
            g = r * _NBUF + s
            in_copy(g, s).wait()
            e = jnp.exp(in_buf[s] * inv_temp)
            ssum = jnp.sum(e, axis=-1, keepdims=True)

            @pl.when(r >= 1)
            def _():
                out_copy(g - _NBUF, s).wait()

            out_buf[s] = e * (1.0 / ssum)
            out_copy(g, s).start()

            @pl.when(g + _NBUF < _NCHUNK)
            def _():
                in_copy(g + _NBUF, s).start()

        return carry

    lax.fori_loop(0, _NROUND, round_body, 0)
    for s in range(_NBUF):
        out_copy(_NCHUNK - _NBUF + s, s).wait()


def kernel(x, temp_log):
    xf = x.reshape(_ROWS, _D)
    tl = temp_log.reshape(1, 1)
    probs = pl.pallas_call(
        _body,
        in_specs=[
            pl.BlockSpec(memory_space=pltpu.MemorySpace.SMEM),
            pl.BlockSpec(memory_space=pltpu.MemorySpace.HBM),
        ],
        out_specs=pl.BlockSpec(memory_space=pltpu.MemorySpace.HBM),
        out_shape=jax.ShapeDtypeStruct((_ROWS, _D), x.dtype),
        scratch_shapes=[
            pltpu.VMEM((_NBUF, _R, _D), jnp.float32),
            pltpu.VMEM((_NBUF, _R, _D), jnp.float32),
            pltpu.SemaphoreType.DMA((_NBUF,)),
            pltpu.SemaphoreType.DMA((_NBUF,)),
        ],
        compiler_params=pltpu.CompilerParams(
            vmem_limit_bytes=62 * 1024 * 1024,
        ),
    )(tl, xf)
    return probs.reshape(x.shape), jnp.exp(temp_log)


# speedup vs baseline: 1.4106x; 1.0002x over previous
import jax
import jax.numpy as jnp
from jax import lax
from jax.experimental import pallas as pl
from jax.experimental.pallas import tpu as pltpu

_B0, _B1, _D = 8, 576, 8192
_ROWS = _B0 * _B1
_R = 96
_NBUF = 6
_NCHUNK = _ROWS // _R
_NROUND = _NCHUNK // _NBUF


def _body(tl_ref, x_hbm, o_hbm, in_buf, out_buf, in_sem, out_sem):
    inv_temp = jnp.exp(-tl_ref[0, 0])

    def in_copy(g, s):
        return pltpu.make_async_copy(
            x_hbm.at[pl.ds(g * _R, _R)], in_buf.at[s], in_sem.at[s])

    def out_copy(g, s):
        return pltpu.make_async_copy(
            out_buf.at[s], o_hbm.at[pl.ds(g * _R, _R)], out_sem.at[s])

    for s in range(_NBUF):
        in_copy(s, s).start()

    def round_body(r, carry):
        for s in range(_NBUF):
            g = r * _NBUF + s
            in_copy(g, s).wait()
            e = jnp.exp(in_buf[s] * inv_temp)
            ssum = jnp.sum(e, axis=-1, keepdims=True)

            @pl.when(r >= 1)
            def _():
                out_copy(g - _NBUF, s).wait()

            out_buf[s] = e * (1.0 / ssum)
            out_copy(g, s).start()

            @pl.when(g + _NBUF < _NCHUNK)
            def _():
                in_copy(g + _NBUF, s).start()

        return carry

    lax.fori_loop(0, _NROUND, round_body, 0)
    for s in range(_NBUF):
        out_copy(_NCHUNK - _NBUF + s, s).wait()


def kernel(x, temp_log):
    xf = x.reshape(_ROWS, _D)
    tl = temp_log.reshape(1, 1)
    probs = pl.pallas_call(
        _body,
        in_specs=[
            pl.BlockSpec(memory_space=pltpu.MemorySpace.SMEM),
            pl.BlockSpec(memory_space=pltpu.MemorySpace.HBM),
        ],
        out_specs=pl.BlockSpec(memory_space=pltpu.MemorySpace.HBM),
        out_shape=jax.ShapeDtypeStruct((_ROWS, _D), x.dtype),
        scratch_shapes=[
            pltpu.VMEM((_NBUF, _R, _D), jnp.float32),
            pltpu.VMEM((_NBUF, _R, _D), jnp.float32),
            pltpu.SemaphoreType.DMA((_NBUF,)),
            pltpu.SemaphoreType.DMA((_NBUF,)),
        ],
        compiler_params=pltpu.CompilerParams(
            vmem_limit_bytes=62 * 1024 * 1024,
        ),
    )(tl, xf)
    return probs.reshape(x.shape), jnp.exp(temp_log)


# speedup vs baseline: 1.4205x; 1.0071x over previous
import jax
import jax.numpy as jnp
from jax import lax
from jax.experimental import pallas as pl
from jax.experimental.pallas import tpu as pltpu

_B0, _B1, _D = 8, 576, 8192
_ROWS = _B0 * _B1
_R = 96
_NBUF = 6
_NCHUNK = _ROWS // _R
_NROUND = _NCHUNK // _NBUF


def _body(tl_ref, x_hbm, o_hbm, t_ref, in_buf, out_buf, in_sem, out_sem):
    t_ref[0, 0] = jnp.exp(tl_ref[0, 0])
    inv_temp = jnp.exp(-tl_ref[0, 0])

    def in_copy(g, s):
        return pltpu.make_async_copy(
            x_hbm.at[pl.ds(g * _R, _R)], in_buf.at[s], in_sem.at[s])

    def out_copy(g, s):
        return pltpu.make_async_copy(
            out_buf.at[s], o_hbm.at[pl.ds(g * _R, _R)], out_sem.at[s])

    for s in range(_NBUF):
        in_copy(s, s).start()

    def round_body(r, carry):
        for s in range(_NBUF):
            g = r * _NBUF + s
            in_copy(g, s).wait()
            e = jnp.exp(in_buf[s] * inv_temp)
            ssum = jnp.sum(e, axis=-1, keepdims=True)

            @pl.when(r >= 1)
            def _():
                out_copy(g - _NBUF, s).wait()

            out_buf[s] = e * (1.0 / ssum)
            out_copy(g, s).start()

            @pl.when(g + _NBUF < _NCHUNK)
            def _():
                in_copy(g + _NBUF, s).start()

        return carry

    lax.fori_loop(0, _NROUND, round_body, 0)
    for s in range(_NBUF):
        out_copy(_NCHUNK - _NBUF + s, s).wait()


def kernel(x, temp_log):
    xf = x.reshape(_ROWS, _D)
    tl = temp_log.reshape(1, 1)
    probs = pl.pallas_call(
        _body,
        in_specs=[
            pl.BlockSpec(memory_space=pltpu.MemorySpace.SMEM),
            pl.BlockSpec(memory_space=pltpu.MemorySpace.HBM),
        ],
        out_specs=[
            pl.BlockSpec(memory_space=pltpu.MemorySpace.HBM),
            pl.BlockSpec(memory_space=pltpu.MemorySpace.SMEM),
        ],
        out_shape=[
            jax.ShapeDtypeStruct((_ROWS, _D), x.dtype),
            jax.ShapeDtypeStruct((1, 1), jnp.float32),
        ],
        scratch_shapes=[
            pltpu.VMEM((_NBUF, _R, _D), jnp.float32),
            pltpu.VMEM((_NBUF, _R, _D), jnp.float32),
            pltpu.SemaphoreType.DMA((_NBUF,)),
            pltpu.SemaphoreType.DMA((_NBUF,)),
        ],
        compiler_params=pltpu.CompilerParams(
            vmem_limit_bytes=62 * 1024 * 1024,
        ),
    )(tl, xf)
    probs, temp = probs
    return probs.reshape(x.shape), temp.reshape(1)
